# baseline (speedup 1.0000x reference)
import jax
import jax.numpy as jnp
from jax import lax
from jax.experimental import pallas as pl
from jax.experimental.pallas import tpu as pltpu
from jax.experimental.pallas import tpu_sc as plsc

B = 16384
D = 64
NC = 2
NS = 16
NW = NC * NS
BPW = B // NW
HB = BPW // 2


def _emb_body(xu_hbm, xp_hbm, xn_hbm, uid2_hbm, iid2_hbm, out_hbm,
              iu_v, ip_v, in_v, tu_v, tpn_v, up_v, pnp_v, blk_v, su, spn):
    c = lax.axis_index("c")
    s = lax.axis_index("s")
    wid = s * NC + c
    base = wid * BPW
    pltpu.sync_copy(xu_hbm.at[pl.ds(base, BPW)], iu_v)
    pltpu.sync_copy(xp_hbm.at[pl.ds(base, BPW)], ip_v)
    pltpu.sync_copy(xn_hbm.at[pl.ds(base, BPW)], in_v)

    def half(h, carry):
        for g in range(HB // 16):
            r = h * HB + g * 16
            tu_v[pl.ds(g * 16, 16)] = lax.shift_right_logical(
                iu_v[pl.ds(r, 16)], 1)
            tpn_v[pl.ds(2 * g * 16, 16)] = lax.shift_right_logical(
                ip_v[pl.ds(r, 16)], 1)
            tpn_v[pl.ds((2 * g + 1) * 16, 16)] = lax.shift_right_logical(
                in_v[pl.ds(r, 16)], 1)
        cu = pltpu.async_copy(uid2_hbm.at[tu_v], up_v, su)
        cpn = pltpu.async_copy(iid2_hbm.at[tpn_v], pnp_v, spn)
        cu.wait()
        cpn.wait()

        def blk(t, carry2):
            r = h * HB + t * 16
            vu = lax.bitwise_and(iu_v[pl.ds(r, 16)], 1) * 64
            vp = lax.bitwise_and(ip_v[pl.ds(r, 16)], 1) * 64
            vn = lax.bitwise_and(in_v[pl.ds(r, 16)], 1) * 64
            for j in range(16):
                urow = t * 16 + j
                prow = (2 * t) * 16 + j
                nrow = (2 * t + 1) * 16 + j
                for k in range(4):
                    ko = k * 16
                    blk_v[3 * j, pl.ds(ko, 16)] = (
                        up_v[urow, pl.ds(vu[j] + ko, 16)])
                    blk_v[3 * j + 1, pl.ds(ko, 16)] = (
                        pnp_v[prow, pl.ds(vp[j] + ko, 16)])
                    blk_v[3 * j + 2, pl.ds(ko, 16)] = (
                        pnp_v[nrow, pl.ds(vn[j] + ko, 16)])
            pltpu.sync_copy(
                blk_v, out_hbm.at[pl.ds(3 * (base + h * HB + t * 16), 48)])
            return carry2

        lax.fori_loop(0, HB // 16, blk, 0)
        return carry

    lax.fori_loop(0, 2, half, 0)


def kernel(x, uid_table, iid_table):
    x = x.astype(jnp.int32)
    xu = x[:, 0]
    xp = x[:, 1]
    xn = x[:, 2]
    uid2 = uid_table.reshape(500000, 128)
    iid2 = iid_table[:1000000].reshape(500000, 128)
    mesh = plsc.VectorSubcoreMesh(core_axis_name="c", subcore_axis_name="s")
    k = pl.kernel(
        _emb_body,
        out_type=jax.ShapeDtypeStruct((3 * B, D), jnp.float32),
        mesh=mesh,
        compiler_params=pltpu.CompilerParams(use_tc_tiling_on_sc=True),
        scratch_types=[
            pltpu.VMEM((BPW,), jnp.int32),
            pltpu.VMEM((BPW,), jnp.int32),
            pltpu.VMEM((BPW,), jnp.int32),
            pltpu.VMEM((HB,), jnp.int32),
            pltpu.VMEM((2 * HB,), jnp.int32),
            pltpu.VMEM((HB, 128), jnp.float32),
            pltpu.VMEM((2 * HB, 128), jnp.float32),
            pltpu.VMEM((48, D), jnp.float32),
            pltpu.SemaphoreType.DMA,
            pltpu.SemaphoreType.DMA,
        ],
    )
    out = k(xu, xp, xn, uid2, iid2)
    return out.reshape(B, 3, D)


# speedup vs baseline: 1.5661x; 1.5661x over previous
import jax
import jax.numpy as jnp
from jax import lax
from jax.experimental import pallas as pl
from jax.experimental.pallas import tpu as pltpu
from jax.experimental.pallas import tpu_sc as plsc

B = 16384
D = 64
NC = 2
NS = 16
NW = NC * NS
BPW = B // NW
HB = BPW // 2


def _emb_body(xu_hbm, xp_hbm, xn_hbm, uid_hbm, iid_hbm, out_hbm,
              iu_v, ip_v, in_v, big_v, sem):
    c = lax.axis_index("c")
    s = lax.axis_index("s")
    wid = s * NC + c
    base = wid * BPW
    pltpu.sync_copy(xu_hbm.at[pl.ds(base, BPW)], iu_v)
    pltpu.sync_copy(xp_hbm.at[pl.ds(base, BPW)], ip_v)
    pltpu.sync_copy(xn_hbm.at[pl.ds(base, BPW)], in_v)

    def half(h, carry):
        @plsc.parallel_loop(0, HB // 16, unroll=2)
        def group(g):
            r = h * HB + g * 16
            vu = iu_v[pl.ds(r, 16)]
            vp = ip_v[pl.ds(r, 16)]
            vn = in_v[pl.ds(r, 16)]
            for j in range(16):
                d = 3 * (g * 16 + j)
                pltpu.async_copy(uid_hbm.at[pl.ds(vu[j], 1)],
                                 big_v.at[pl.ds(d, 1)], sem)
                pltpu.async_copy(iid_hbm.at[pl.ds(vp[j], 1)],
                                 big_v.at[pl.ds(d + 1, 1)], sem)
                pltpu.async_copy(iid_hbm.at[pl.ds(vn[j], 1)],
                                 big_v.at[pl.ds(d + 2, 1)], sem)

        pltpu.make_async_copy(uid_hbm.at[pl.ds(0, 3 * HB)], big_v, sem).wait()
        pltpu.sync_copy(big_v, out_hbm.at[pl.ds(3 * (base + h * HB), 3 * HB)])
        return carry

    lax.fori_loop(0, 2, half, 0)


def kernel(x, uid_table, iid_table):
    x = x.astype(jnp.int32)
    xu = x[:, 0]
    xp = x[:, 1]
    xn = x[:, 2]
    mesh = plsc.VectorSubcoreMesh(core_axis_name="c", subcore_axis_name="s")
    k = pl.kernel(
        _emb_body,
        out_type=jax.ShapeDtypeStruct((3 * B, D), jnp.float32),
        mesh=mesh,
        compiler_params=pltpu.CompilerParams(use_tc_tiling_on_sc=True),
        scratch_types=[
            pltpu.VMEM((BPW,), jnp.int32),
            pltpu.VMEM((BPW,), jnp.int32),
            pltpu.VMEM((BPW,), jnp.int32),
            pltpu.VMEM((3 * HB, D), jnp.float32),
            pltpu.SemaphoreType.DMA,
        ],
    )
    out = k(xu, xp, xn, uid_table, iid_table)
    return out.reshape(B, 3, D)
